# baseline (speedup 1.0000x reference)
import jax
import jax.numpy as jnp
from jax.experimental import pallas as pl
from jax.experimental.pallas import tpu as pltpu

NH = 8
HD = 96
K_TOP = 4
NW = 7
RS = 16
NREG = NW * NW
RS2 = RS * RS
D = 768
C_IN = 384
H = 112
SCALE = HD ** -0.5
F32 = jnp.float32
BF16 = jnp.bfloat16


def _k1_body(zf_ref, w2_ref, b2_ref, wqkv_ref, bqkv_ref, qkv_ref, zfpool_ref):
    zf = zf_ref[:].reshape(RS2, C_IN)
    zfpool_ref[:] = jnp.mean(zf, axis=0).reshape(1, 1, C_IN)
    x = jnp.dot(zf.astype(BF16), w2_ref[:], preferred_element_type=F32)
    x = (x + b2_ref[:]).astype(BF16)
    qkv = jnp.dot(x, wqkv_ref[:], preferred_element_type=F32) + bqkv_ref[:]
    qkv_ref[:] = qkv.astype(BF16).reshape(RS, RS, 3 * D)


def _k2_body(zfpool_ref, w2_ref, b2_ref, wq_ref, bq_ref, wk_ref, bk_ref,
             idx_ref):
    hi = jax.lax.Precision.HIGHEST
    zp = zfpool_ref[:].reshape(NREG, C_IN)
    xp = jnp.dot(zp, w2_ref[:], precision=hi,
                 preferred_element_type=F32) + b2_ref[:]
    qr = jnp.dot(xp, wq_ref[:], precision=hi,
                 preferred_element_type=F32) + bq_ref[:]
    kr = jnp.dot(xp, wk_ref[:], precision=hi,
                 preferred_element_type=F32) + bk_ref[:]
    a = jax.lax.dot_general(qr, kr, (((1,), (1,)), ((), ())), precision=hi,
                            preferred_element_type=F32)
    iota = jax.lax.broadcasted_iota(jnp.int32, (NREG, NREG), 1)
    cols = []
    for _ in range(K_TOP):
        m = jnp.max(a, axis=1, keepdims=True)
        amax = jnp.min(jnp.where(a == m, iota, NREG), axis=1, keepdims=True)
        cols.append(amax)
        a = jnp.where(iota == amax, -jnp.inf, a)
    idx_ref[:] = jnp.concatenate(cols, axis=1)


def _k3_body(idx_ref, q_ref, k0_ref, k1_ref, k2_ref, k3_ref,
             v0_ref, v1_ref, v2_ref, v3_ref, out_ref):
    del idx_ref
    q = q_ref[:].reshape(RS2, D)
    k = jnp.concatenate([r[:].reshape(RS2, D)
                         for r in (k0_ref, k1_ref, k2_ref, k3_ref)], axis=0)
    v = jnp.concatenate([r[:].reshape(RS2, D)
                         for r in (v0_ref, v1_ref, v2_ref, v3_ref)], axis=0)
    outs = []
    for h in range(NH):
        sl = slice(h * HD, (h + 1) * HD)
        qh, kh, vh = q[:, sl], k[:, sl], v[:, sl]
        lg = jax.lax.dot_general(qh, kh, (((1,), (1,)), ((), ())),
                                 preferred_element_type=F32) * SCALE
        m = jnp.max(lg, axis=1, keepdims=True)
        e = jnp.exp(lg - m)
        p = (e / jnp.sum(e, axis=1, keepdims=True)).astype(BF16)
        outs.append(jnp.dot(p, vh, preferred_element_type=F32))
    out_ref[:] = jnp.concatenate(outs, axis=1).astype(BF16).reshape(RS, RS, D)


def _k4_body(vpad_ref, w_ref, b_ref, out_ref):
    cb = b_ref.shape[1]
    acc = jnp.broadcast_to(b_ref[:].reshape(1, 1, cb), (H, H, cb))
    for dy in range(5):
        for dx in range(5):
            w = w_ref[dy * 5 + dx: dy * 5 + dx + 1, :].reshape(1, 1, cb)
            acc = acc + vpad_ref[dy:dy + H, dx:dx + H, :].astype(F32) * w
    out_ref[:] = acc.astype(BF16)


def _k5_body(a_ref, l_ref, zf_ref, wout_ref, bout_ref, w3_ref, b3_ref,
             out_ref):
    o = (a_ref[:].astype(F32) + l_ref[:].astype(F32)).astype(BF16)
    y = jnp.dot(o, wout_ref[:], preferred_element_type=F32) + bout_ref[:]
    z = jnp.dot(y.astype(BF16), w3_ref[:], preferred_element_type=F32) + b3_ref[:]
    out_ref[:] = z + zf_ref[:]


def kernel(zf, W2, b2, Wqkv, bqkv, Wlepe, blepe, Wout, bout, W3, b3):
    zf_t = jnp.transpose(zf[0], (1, 2, 0))

    qkv_t, zfpool = pl.pallas_call(
        _k1_body,
        grid=(NW, NW),
        compiler_params=pltpu.CompilerParams(
            dimension_semantics=("parallel", "parallel")),
        in_specs=[
            pl.BlockSpec((RS, RS, C_IN), lambda i, j: (i, j, 0)),
            pl.BlockSpec((C_IN, D), lambda i, j: (0, 0)),
            pl.BlockSpec((1, D), lambda i, j: (0, 0)),
            pl.BlockSpec((D, 3 * D), lambda i, j: (0, 0)),
            pl.BlockSpec((1, 3 * D), lambda i, j: (0, 0)),
        ],
        out_specs=[
            pl.BlockSpec((RS, RS, 3 * D), lambda i, j: (i, j, 0)),
            pl.BlockSpec((1, 1, C_IN), lambda i, j: (i * NW + j, 0, 0)),
        ],
        out_shape=[
            jax.ShapeDtypeStruct((H, H, 3 * D), BF16),
            jax.ShapeDtypeStruct((NREG, 1, C_IN), F32),
        ],
    )(zf_t, W2.T.astype(BF16), b2[None], Wqkv.T.astype(BF16), bqkv[None])

    idx = pl.pallas_call(
        _k2_body,
        out_shape=jax.ShapeDtypeStruct((NREG, K_TOP), jnp.int32),
    )(zfpool, W2.T, b2[None], Wqkv[:D].T, bqkv[:D][None],
      Wqkv[D:2 * D].T, bqkv[D:2 * D][None])

    def _qmap(n, idx):
        return (n // NW, n % NW, 0)

    def _kmap(t):
        return lambda n, idx: (idx[n, t] // NW, idx[n, t] % NW, 1)

    def _vmap(t):
        return lambda n, idx: (idx[n, t] // NW, idx[n, t] % NW, 2)

    blk = pl.BlockSpec((RS, RS, D), _qmap)
    attn_out = pl.pallas_call(
        _k3_body,
        grid_spec=pltpu.PrefetchScalarGridSpec(
            num_scalar_prefetch=1,
            grid=(NREG,),
            in_specs=[blk]
            + [pl.BlockSpec((RS, RS, D), _kmap(t)) for t in range(K_TOP)]
            + [pl.BlockSpec((RS, RS, D), _vmap(t)) for t in range(K_TOP)],
            out_specs=pl.BlockSpec((RS, RS, D), _qmap),
        ),
        compiler_params=pltpu.CompilerParams(
            dimension_semantics=("parallel",)),
        out_shape=jax.ShapeDtypeStruct((H, H, D), BF16),
    )(idx, *([qkv_t] * 9))

    vpad = jnp.pad(qkv_t[:, :, 2 * D:], ((2, 2), (2, 2), (0, 0)))
    wl = jnp.pad(Wlepe.reshape(D, 25).T, ((0, 7), (0, 0)))
    CB = 128
    lepe = pl.pallas_call(
        _k4_body,
        grid=(D // CB,),
        compiler_params=pltpu.CompilerParams(
            dimension_semantics=("parallel",)),
        in_specs=[
            pl.BlockSpec((H + 4, H + 4, CB), lambda c: (0, 0, c)),
            pl.BlockSpec((32, CB), lambda c: (0, c)),
            pl.BlockSpec((1, CB), lambda c: (0, c)),
        ],
        out_specs=pl.BlockSpec((H, H, CB), lambda c: (0, 0, c)),
        out_shape=jax.ShapeDtypeStruct((H, H, D), BF16),
    )(vpad, wl, blepe[None])

    ntok = H * H
    TB = 256
    out_flat = pl.pallas_call(
        _k5_body,
        grid=(ntok // TB,),
        compiler_params=pltpu.CompilerParams(
            dimension_semantics=("parallel",)),
        in_specs=[
            pl.BlockSpec((TB, D), lambda n: (n, 0)),
            pl.BlockSpec((TB, D), lambda n: (n, 0)),
            pl.BlockSpec((TB, C_IN), lambda n: (n, 0)),
            pl.BlockSpec((D, D), lambda n: (0, 0)),
            pl.BlockSpec((1, D), lambda n: (0, 0)),
            pl.BlockSpec((D, C_IN), lambda n: (0, 0)),
            pl.BlockSpec((1, C_IN), lambda n: (0, 0)),
        ],
        out_specs=pl.BlockSpec((TB, C_IN), lambda n: (n, 0)),
        out_shape=jax.ShapeDtypeStruct((ntok, C_IN), F32),
    )(attn_out.reshape(ntok, D), lepe.reshape(ntok, D),
      zf_t.reshape(ntok, C_IN), Wout.T.astype(BF16), bout[None],
      W3.T.astype(BF16), b3[None])

    return jnp.transpose(out_flat.reshape(H, H, C_IN), (2, 0, 1))[None]


# speedup vs baseline: 1.0237x; 1.0237x over previous
import jax
import jax.numpy as jnp
from jax.experimental import pallas as pl
from jax.experimental.pallas import tpu as pltpu

NH = 8
HD = 96
HP = 128
K_TOP = 4
NW = 7
RS = 16
NREG = NW * NW
RS2 = RS * RS
D = 768
DP = NH * HP
C_IN = 384
H = 112
SCALE = HD ** -0.5
F32 = jnp.float32
BF16 = jnp.bfloat16


def _k1_body(zf_ref, w2_ref, b2_ref, wqkv_ref, bqkv_ref, qkv_ref, zfpool_ref):
    zf = zf_ref[:].reshape(RS2, C_IN)
    zfpool_ref[:] = jnp.mean(zf, axis=0).reshape(1, 1, C_IN)
    x = jnp.dot(zf.astype(BF16), w2_ref[:], preferred_element_type=F32)
    x = (x + b2_ref[:]).astype(BF16)
    qkv = jnp.dot(x, wqkv_ref[:], preferred_element_type=F32) + bqkv_ref[:]
    qkv_ref[:] = qkv.astype(BF16).reshape(RS, RS, 3 * DP)


def _k2_body(zfpool_ref, w2_ref, b2_ref, wq_ref, bq_ref, wk_ref, bk_ref,
             idx_ref):
    hi = jax.lax.Precision.HIGHEST
    zp = zfpool_ref[:].reshape(NREG, C_IN)
    xp = jnp.dot(zp, w2_ref[:], precision=hi,
                 preferred_element_type=F32) + b2_ref[:]
    qr = jnp.dot(xp, wq_ref[:], precision=hi,
                 preferred_element_type=F32) + bq_ref[:]
    kr = jnp.dot(xp, wk_ref[:], precision=hi,
                 preferred_element_type=F32) + bk_ref[:]
    a = jax.lax.dot_general(qr, kr, (((1,), (1,)), ((), ())), precision=hi,
                            preferred_element_type=F32)
    iota = jax.lax.broadcasted_iota(jnp.int32, (NREG, NREG), 1)
    cols = []
    for _ in range(K_TOP):
        m = jnp.max(a, axis=1, keepdims=True)
        amax = jnp.min(jnp.where(a == m, iota, NREG), axis=1, keepdims=True)
        cols.append(amax)
        a = jnp.where(iota == amax, -jnp.inf, a)
    idx_ref[:] = jnp.concatenate(cols, axis=1)


def _k3_body(idx_ref, q_ref, k0_ref, k1_ref, k2_ref, k3_ref,
             v0_ref, v1_ref, v2_ref, v3_ref, out_ref):
    del idx_ref
    q = q_ref[:].reshape(RS2, DP)
    ks = [r[:].reshape(RS2, DP) for r in (k0_ref, k1_ref, k2_ref, k3_ref)]
    vs = [r[:].reshape(RS2, DP) for r in (v0_ref, v1_ref, v2_ref, v3_ref)]
    outs = []
    for h in range(NH):
        sl = slice(h * HP, (h + 1) * HP)
        qh = q[:, sl]
        oacc = jnp.zeros((RS2, HP), F32)
        for t in range(K_TOP):
            lg = jax.lax.dot_general(qh, ks[t][:, sl], (((1,), (1,)), ((), ())),
                                     preferred_element_type=F32)
            e = jnp.exp(lg).astype(BF16)
            oacc = oacc + jnp.dot(e, vs[t][:, sl],
                                  preferred_element_type=F32)
        outs.append((oacc / oacc[:, HD:HD + 1]).astype(BF16))
    out_ref[:] = jnp.concatenate(outs, axis=1).reshape(RS, RS, DP)


def _k4_body(vpad_ref, w_ref, b_ref, out_ref):
    cb = b_ref.shape[1]
    acc = jnp.broadcast_to(b_ref[:].reshape(1, 1, cb), (H, H, cb))
    for dy in range(5):
        for dx in range(5):
            w = w_ref[dy * 5 + dx: dy * 5 + dx + 1, :].reshape(1, 1, cb)
            acc = acc + vpad_ref[dy:dy + H, dx:dx + H, :].astype(F32) * w
    out_ref[:] = acc.astype(BF16)


def _k5_body(a_ref, l_ref, zf_ref, wout_ref, bout_ref, w3_ref, b3_ref,
             out_ref):
    o = (a_ref[:].astype(F32) + l_ref[:].astype(F32)).astype(BF16)
    y = jnp.dot(o, wout_ref[:], preferred_element_type=F32) + bout_ref[:]
    z = jnp.dot(y.astype(BF16), w3_ref[:], preferred_element_type=F32) + b3_ref[:]
    out_ref[:] = z + zf_ref[:]


def _pad_heads(w):
    r = w.shape[0]
    return jnp.pad(w.reshape(r, NH, HD), ((0, 0), (0, 0), (0, HP - HD))
                   ).reshape(r, DP)


def kernel(zf, W2, b2, Wqkv, bqkv, Wlepe, blepe, Wout, bout, W3, b3):
    zf_t = jnp.transpose(zf[0], (1, 2, 0))

    wq, wk, wv = Wqkv[:D].T, Wqkv[D:2 * D].T, Wqkv[2 * D:].T
    wqkv_pad = jnp.concatenate(
        [_pad_heads(wq * SCALE), _pad_heads(wk), _pad_heads(wv)], axis=1)
    bq, bk, bv = bqkv[:D], bqkv[D:2 * D], bqkv[2 * D:]
    bv_pad = _pad_heads(bv[None])[0]
    bv_pad = bv_pad.at[HD::HP].set(1.0)
    bqkv_pad = jnp.concatenate(
        [_pad_heads(bq[None] * SCALE)[0], _pad_heads(bk[None])[0], bv_pad])

    qkv_t, zfpool = pl.pallas_call(
        _k1_body,
        grid=(NW, NW),
        compiler_params=pltpu.CompilerParams(
            dimension_semantics=("parallel", "parallel")),
        in_specs=[
            pl.BlockSpec((RS, RS, C_IN), lambda i, j: (i, j, 0)),
            pl.BlockSpec((C_IN, D), lambda i, j: (0, 0)),
            pl.BlockSpec((1, D), lambda i, j: (0, 0)),
            pl.BlockSpec((D, 3 * DP), lambda i, j: (0, 0)),
            pl.BlockSpec((1, 3 * DP), lambda i, j: (0, 0)),
        ],
        out_specs=[
            pl.BlockSpec((RS, RS, 3 * DP), lambda i, j: (i, j, 0)),
            pl.BlockSpec((1, 1, C_IN), lambda i, j: (i * NW + j, 0, 0)),
        ],
        out_shape=[
            jax.ShapeDtypeStruct((H, H, 3 * DP), BF16),
            jax.ShapeDtypeStruct((NREG, 1, C_IN), F32),
        ],
    )(zf_t, W2.T.astype(BF16), b2[None], wqkv_pad.astype(BF16),
      bqkv_pad[None])

    idx = pl.pallas_call(
        _k2_body,
        out_shape=jax.ShapeDtypeStruct((NREG, K_TOP), jnp.int32),
    )(zfpool, W2.T, b2[None], wq, bq[None], wk, bk[None])

    def _qmap(n, idx):
        return (n // NW, n % NW, 0)

    def _kmap(t):
        return lambda n, idx: (idx[n, t] // NW, idx[n, t] % NW, 1)

    def _vmap(t):
        return lambda n, idx: (idx[n, t] // NW, idx[n, t] % NW, 2)

    blk = pl.BlockSpec((RS, RS, DP), _qmap)
    attn_out = pl.pallas_call(
        _k3_body,
        grid_spec=pltpu.PrefetchScalarGridSpec(
            num_scalar_prefetch=1,
            grid=(NREG,),
            in_specs=[blk]
            + [pl.BlockSpec((RS, RS, DP), _kmap(t)) for t in range(K_TOP)]
            + [pl.BlockSpec((RS, RS, DP), _vmap(t)) for t in range(K_TOP)],
            out_specs=pl.BlockSpec((RS, RS, DP), _qmap),
        ),
        compiler_params=pltpu.CompilerParams(
            dimension_semantics=("parallel",)),
        out_shape=jax.ShapeDtypeStruct((H, H, DP), BF16),
    )(idx, *([qkv_t] * 9))

    vpad = jnp.pad(qkv_t[:, :, 2 * DP:], ((2, 2), (2, 2), (0, 0)))
    wl = jnp.pad(_pad_heads(Wlepe.reshape(D, 25).T), ((0, 7), (0, 0)))
    bl_pad = _pad_heads(blepe[None])
    CB = 128
    lepe = pl.pallas_call(
        _k4_body,
        grid=(DP // CB,),
        compiler_params=pltpu.CompilerParams(
            dimension_semantics=("parallel",)),
        in_specs=[
            pl.BlockSpec((H + 4, H + 4, CB), lambda c: (0, 0, c)),
            pl.BlockSpec((32, CB), lambda c: (0, c)),
            pl.BlockSpec((1, CB), lambda c: (0, c)),
        ],
        out_specs=pl.BlockSpec((H, H, CB), lambda c: (0, 0, c)),
        out_shape=jax.ShapeDtypeStruct((H, H, DP), BF16),
    )(vpad, wl, bl_pad)

    wout_pad = jnp.pad(Wout.T.reshape(NH, HD, D), ((0, 0), (0, HP - HD),
                                                   (0, 0))).reshape(DP, D)

    ntok = H * H
    TB = 256
    out_flat = pl.pallas_call(
        _k5_body,
        grid=(ntok // TB,),
        compiler_params=pltpu.CompilerParams(
            dimension_semantics=("parallel",)),
        in_specs=[
            pl.BlockSpec((TB, DP), lambda n: (n, 0)),
            pl.BlockSpec((TB, DP), lambda n: (n, 0)),
            pl.BlockSpec((TB, C_IN), lambda n: (n, 0)),
            pl.BlockSpec((DP, D), lambda n: (0, 0)),
            pl.BlockSpec((1, D), lambda n: (0, 0)),
            pl.BlockSpec((D, C_IN), lambda n: (0, 0)),
            pl.BlockSpec((1, C_IN), lambda n: (0, 0)),
        ],
        out_specs=pl.BlockSpec((TB, C_IN), lambda n: (n, 0)),
        out_shape=jax.ShapeDtypeStruct((ntok, C_IN), F32),
    )(attn_out.reshape(ntok, DP), lepe.reshape(ntok, DP),
      zf_t.reshape(ntok, C_IN), wout_pad.astype(BF16), bout[None],
      W3.T.astype(BF16), b3[None])

    return jnp.transpose(out_flat.reshape(H, H, C_IN), (2, 0, 1))[None]


# speedup vs baseline: 1.2151x; 1.1870x over previous
import jax
import jax.numpy as jnp
from jax.experimental import pallas as pl
from jax.experimental.pallas import tpu as pltpu

NH = 8
HD = 96
HP = 128
K_TOP = 4
NW = 7
RS = 16
NREG = NW * NW
RS2 = RS * RS
D = 768
DP = NH * HP
C_IN = 384
H = 112
SCALE = HD ** -0.5
F32 = jnp.float32
BF16 = jnp.bfloat16
FP8 = jnp.float8_e4m3fn
WS = 16.0
WS_INV = 1.0 / WS


def _k1_body(zf_ref, w2_ref, b2_ref, wqkv_ref, bqkv_ref, scl_ref,
             qk_ref, v_ref, zfpool_ref):
    zf = zf_ref[:].reshape(RS2, C_IN)
    zfpool_ref[:] = jnp.mean(zf, axis=0).reshape(1, 1, C_IN)
    x = jnp.dot(zf.astype(FP8), w2_ref[:], preferred_element_type=F32)
    x = (x * WS_INV + b2_ref[:]).astype(FP8)
    qkv = jnp.dot(x, wqkv_ref[:], preferred_element_type=F32)
    qkv = (qkv * scl_ref[:] + bqkv_ref[:]).astype(FP8)
    qk_ref[:] = qkv[:, :2 * DP].reshape(RS, RS, 2 * DP)
    v_ref[:] = qkv[:, 2 * DP:].reshape(RS, RS, DP)


def _k2_body(zfpool_ref, w2_ref, b2_ref, wq_ref, bq_ref, wk_ref, bk_ref,
             idx_ref):
    hi = jax.lax.Precision.HIGHEST
    zp = zfpool_ref[:].reshape(NREG, C_IN)
    xp = jnp.dot(zp, w2_ref[:], precision=hi,
                 preferred_element_type=F32) + b2_ref[:]
    qr = jnp.dot(xp, wq_ref[:], precision=hi,
                 preferred_element_type=F32) + bq_ref[:]
    kr = jnp.dot(xp, wk_ref[:], precision=hi,
                 preferred_element_type=F32) + bk_ref[:]
    a = jax.lax.dot_general(qr, kr, (((1,), (1,)), ((), ())), precision=hi,
                            preferred_element_type=F32)
    iota = jax.lax.broadcasted_iota(jnp.int32, (NREG, NREG), 1)
    cols = []
    for _ in range(K_TOP):
        m = jnp.max(a, axis=1, keepdims=True)
        amax = jnp.min(jnp.where(a == m, iota, NREG), axis=1, keepdims=True)
        cols.append(amax)
        a = jnp.where(iota == amax, -jnp.inf, a)
    idx_ref[:] = jnp.concatenate(cols, axis=1)


def _k3_body(idx_ref, q_ref, k0_ref, k1_ref, k2_ref, k3_ref,
             v0_ref, v1_ref, v2_ref, v3_ref, out_ref):
    del idx_ref
    q = q_ref[:].reshape(RS2, DP)
    ks = [r[:].reshape(RS2, DP) for r in (k0_ref, k1_ref, k2_ref, k3_ref)]
    vs = [r[:].reshape(RS2, DP) for r in (v0_ref, v1_ref, v2_ref, v3_ref)]
    outs = []
    for h in range(NH):
        sl = slice(h * HP, (h + 1) * HP)
        qh = q[:, sl]
        oacc = jnp.zeros((RS2, HP), F32)
        for t in range(K_TOP):
            lg = jax.lax.dot_general(qh, ks[t][:, sl], (((1,), (1,)), ((), ())),
                                     preferred_element_type=F32)
            e = jnp.exp(lg).astype(FP8)
            oacc = oacc + jnp.dot(e, vs[t][:, sl],
                                  preferred_element_type=F32)
        outs.append((oacc / oacc[:, HD:HD + 1]).astype(FP8))
    out_ref[:] = jnp.concatenate(outs, axis=1).reshape(RS, RS, DP)


def _k4_body(vpad_ref, w_ref, b_ref, out_ref):
    cb = b_ref.shape[1]
    acc = jnp.broadcast_to(b_ref[:].reshape(1, 1, cb), (H, H, cb))
    for dy in range(5):
        for dx in range(5):
            w = w_ref[dy * 5 + dx: dy * 5 + dx + 1, :].reshape(1, 1, cb)
            acc = acc + vpad_ref[dy:dy + H, dx:dx + H, :].astype(F32) * w
    out_ref[:] = acc.astype(FP8)


def _k5_body(a_ref, l_ref, zf_ref, wout_ref, bout_ref, w3_ref, b3_ref,
             out_ref):
    o = (a_ref[:].astype(F32) + l_ref[:].astype(F32)).astype(FP8)
    y = jnp.dot(o, wout_ref[:], preferred_element_type=F32)
    y = (y * WS_INV + bout_ref[:]).astype(FP8)
    z = jnp.dot(y, w3_ref[:], preferred_element_type=F32)
    out_ref[:] = z * WS_INV + b3_ref[:] + zf_ref[:]


def _pad_heads(w):
    r = w.shape[0]
    return jnp.pad(w.reshape(r, NH, HD), ((0, 0), (0, 0), (0, HP - HD))
                   ).reshape(r, DP)


def kernel(zf, W2, b2, Wqkv, bqkv, Wlepe, blepe, Wout, bout, W3, b3):
    zf_t = jnp.transpose(zf[0], (1, 2, 0))

    wq, wk, wv = Wqkv[:D].T, Wqkv[D:2 * D].T, Wqkv[2 * D:].T
    wqkv_pad = jnp.concatenate(
        [_pad_heads(wq), _pad_heads(wk), _pad_heads(wv)], axis=1) * WS
    bq, bk, bv = bqkv[:D], bqkv[D:2 * D], bqkv[2 * D:]
    bv_pad = _pad_heads(bv[None])[0]
    bv_pad = bv_pad.at[HD::HP].set(1.0)
    bqkv_pad = jnp.concatenate(
        [_pad_heads(bq[None] * SCALE)[0], _pad_heads(bk[None])[0], bv_pad])
    scl = jnp.concatenate([jnp.full((DP,), SCALE * WS_INV),
                           jnp.full((2 * DP,), WS_INV)])[None]

    qk_t, v_t, zfpool = pl.pallas_call(
        _k1_body,
        grid=(NW, NW),
        compiler_params=pltpu.CompilerParams(
            dimension_semantics=("parallel", "parallel")),
        in_specs=[
            pl.BlockSpec((RS, RS, C_IN), lambda i, j: (i, j, 0)),
            pl.BlockSpec((C_IN, D), lambda i, j: (0, 0)),
            pl.BlockSpec((1, D), lambda i, j: (0, 0)),
            pl.BlockSpec((D, 3 * DP), lambda i, j: (0, 0)),
            pl.BlockSpec((1, 3 * DP), lambda i, j: (0, 0)),
            pl.BlockSpec((1, 3 * DP), lambda i, j: (0, 0)),
        ],
        out_specs=[
            pl.BlockSpec((RS, RS, 2 * DP), lambda i, j: (i, j, 0)),
            pl.BlockSpec((RS, RS, DP), lambda i, j: (i, j, 0)),
            pl.BlockSpec((1, 1, C_IN), lambda i, j: (i * NW + j, 0, 0)),
        ],
        out_shape=[
            jax.ShapeDtypeStruct((H, H, 2 * DP), FP8),
            jax.ShapeDtypeStruct((H, H, DP), FP8),
            jax.ShapeDtypeStruct((NREG, 1, C_IN), F32),
        ],
    )(zf_t, (W2.T * WS).astype(FP8), b2[None], wqkv_pad.astype(FP8),
      bqkv_pad[None], scl)

    idx = pl.pallas_call(
        _k2_body,
        out_shape=jax.ShapeDtypeStruct((NREG, K_TOP), jnp.int32),
    )(zfpool, W2.T, b2[None], wq, bq[None], wk, bk[None])

    def _qmap(n, idx):
        return (n // NW, n % NW, 0)

    def _kmap(t):
        return lambda n, idx: (idx[n, t] // NW, idx[n, t] % NW, 1)

    def _vmap(t):
        return lambda n, idx: (idx[n, t] // NW, idx[n, t] % NW, 0)

    attn_out = pl.pallas_call(
        _k3_body,
        grid_spec=pltpu.PrefetchScalarGridSpec(
            num_scalar_prefetch=1,
            grid=(NREG,),
            in_specs=[pl.BlockSpec((RS, RS, DP), _qmap)]
            + [pl.BlockSpec((RS, RS, DP), _kmap(t)) for t in range(K_TOP)]
            + [pl.BlockSpec((RS, RS, DP), _vmap(t)) for t in range(K_TOP)],
            out_specs=pl.BlockSpec((RS, RS, DP), _qmap),
        ),
        compiler_params=pltpu.CompilerParams(
            dimension_semantics=("parallel",)),
        out_shape=jax.ShapeDtypeStruct((H, H, DP), FP8),
    )(idx, *([qk_t] * 5), *([v_t] * 4))

    vpad = jnp.pad(v_t, ((2, 2), (2, 2), (0, 0)))
    wl = jnp.pad(_pad_heads(Wlepe.reshape(D, 25).T), ((0, 7), (0, 0)))
    bl_pad = _pad_heads(blepe[None])
    CB = 128
    lepe = pl.pallas_call(
        _k4_body,
        grid=(DP // CB,),
        compiler_params=pltpu.CompilerParams(
            dimension_semantics=("parallel",)),
        in_specs=[
            pl.BlockSpec((H + 4, H + 4, CB), lambda c: (0, 0, c)),
            pl.BlockSpec((32, CB), lambda c: (0, c)),
            pl.BlockSpec((1, CB), lambda c: (0, c)),
        ],
        out_specs=pl.BlockSpec((H, H, CB), lambda c: (0, 0, c)),
        out_shape=jax.ShapeDtypeStruct((H, H, DP), FP8),
    )(vpad, wl, bl_pad)

    wout_pad = jnp.pad(Wout.T.reshape(NH, HD, D), ((0, 0), (0, HP - HD),
                                                   (0, 0))).reshape(DP, D)

    ntok = H * H
    TB = 256
    out_flat = pl.pallas_call(
        _k5_body,
        grid=(ntok // TB,),
        compiler_params=pltpu.CompilerParams(
            dimension_semantics=("parallel",)),
        in_specs=[
            pl.BlockSpec((TB, DP), lambda n: (n, 0)),
            pl.BlockSpec((TB, DP), lambda n: (n, 0)),
            pl.BlockSpec((TB, C_IN), lambda n: (n, 0)),
            pl.BlockSpec((DP, D), lambda n: (0, 0)),
            pl.BlockSpec((1, D), lambda n: (0, 0)),
            pl.BlockSpec((D, C_IN), lambda n: (0, 0)),
            pl.BlockSpec((1, C_IN), lambda n: (0, 0)),
        ],
        out_specs=pl.BlockSpec((TB, C_IN), lambda n: (n, 0)),
        out_shape=jax.ShapeDtypeStruct((ntok, C_IN), F32),
    )(attn_out.reshape(ntok, DP), lepe.reshape(ntok, DP),
      zf_t.reshape(ntok, C_IN), (wout_pad * WS).astype(FP8), bout[None],
      (W3.T * WS).astype(FP8), b3[None])

    return jnp.transpose(out_flat.reshape(H, H, C_IN), (2, 0, 1))[None]
